# speedup vs baseline: 1.2485x; 1.2485x over previous
import jax
import jax.numpy as jnp
from jax.experimental import pallas as pl
from jax.experimental.pallas import tpu as pltpu


def _cholec_body(gp_ref, gt_ref, o_ref):
    gp = gp_ref[0]
    gt = gt_ref[0]

    gp_b = (gp != 0).astype(jnp.bfloat16)
    gt_b = (gt != 0).astype(jnp.bfloat16)

    inters = jax.lax.dot_general(
        gp_b, gt_b, (((1,), (1,)), ((), ())),
        preferred_element_type=jnp.float32)

    gt_area = jnp.sum(gt_b.astype(jnp.float32), axis=1)
    safe_area = jnp.where(gt_area > 0.0, gt_area, 1.0)
    iogs = jnp.where(gt_area[None, :] > 0.0, inters / safe_area[None, :], 0.0)
    iog_max = jnp.max(iogs, axis=1)

    gp_f = (gp != 0).astype(jnp.float32)
    pas = jnp.sum(gp_f * iog_max[:, None], axis=0)
    cover = jnp.sum(gp_f, axis=0)
    o_ref[0, 0] = jnp.where(cover > 0.0, pas / cover, 0.0)


def kernel(groups_pred, groups_true):
    N, P, H, W = groups_pred.shape
    T = groups_true.shape[1]
    HW = H * W
    gp3 = groups_pred.reshape(N, P, HW)
    gt3 = groups_true.reshape(N, T, HW)

    out = pl.pallas_call(
        _cholec_body,
        grid=(N,),
        in_specs=[
            pl.BlockSpec((1, P, HW), lambda n: (n, 0, 0)),
            pl.BlockSpec((1, T, HW), lambda n: (n, 0, 0)),
        ],
        out_specs=pl.BlockSpec((1, 1, HW), lambda n: (n, 0, 0)),
        out_shape=jax.ShapeDtypeStruct((N, 1, HW), jnp.float32),
        compiler_params=pltpu.CompilerParams(
            dimension_semantics=("parallel",),
            vmem_limit_bytes=56 * 1024 * 1024,
        ),
        name="cholec_metric",
    )(gp3, gt3)
    return out.reshape(N, H, W)


# speedup vs baseline: 4.2226x; 3.3822x over previous
import jax
import jax.numpy as jnp
from jax import lax
from jax.experimental import pallas as pl
from jax.experimental.pallas import tpu as pltpu


def _cholec_body(gp_ref, gt_ref, o_ref):
    gp = gp_ref[0]
    gt = gt_ref[0]
    P, H, W = gp.shape
    T = gt.shape[0]

    gp_m = (gp != 0).astype(jnp.float32)
    gt_m = (gt != 0).astype(jnp.float32)

    acc = jnp.zeros((P * 8, T * 8), jnp.float32)
    for c in range(H // 8):
        a_c = gp_m[:, 8 * c:8 * c + 8, :].reshape(P * 8, W).astype(jnp.bfloat16)
        b_c = gt_m[:, 8 * c:8 * c + 8, :].reshape(T * 8, W).astype(jnp.bfloat16)
        acc = acc + lax.dot_general(
            a_c, b_c, (((1,), (1,)), ((), ())),
            preferred_element_type=jnp.float32)

    ph = lax.broadcasted_iota(jnp.int32, (P * 8, T * 8), 0) % 8
    th = lax.broadcasted_iota(jnp.int32, (P * 8, T * 8), 1) % 8
    accm = jnp.where(ph == th, acc, 0.0)
    s2 = accm.reshape(P, 8, T * 8).sum(axis=1)
    fold = (lax.broadcasted_iota(jnp.int32, (T * 8, T), 0) // 8
            == lax.broadcasted_iota(jnp.int32, (T * 8, T), 1)
            ).astype(jnp.float32)
    inters = lax.dot_general(
        s2, fold, (((1,), (0,)), ((), ())),
        preferred_element_type=jnp.float32)

    area = jnp.sum(gt_m, axis=(1, 2))
    safe = jnp.where(area > 0.0, area, 1.0)
    iogs = jnp.where(area[None, :] > 0.0, inters / safe[None, :], 0.0)
    iog_max = jnp.max(iogs, axis=1)

    pas = jnp.sum(gp_m * iog_max[:, None, None], axis=0)
    cover = jnp.sum(gp_m, axis=0)
    o_ref[0] = jnp.where(cover > 0.0, pas / cover, 0.0)


def kernel(groups_pred, groups_true):
    N, P, H, W = groups_pred.shape
    T = groups_true.shape[1]

    return pl.pallas_call(
        _cholec_body,
        grid=(N,),
        in_specs=[
            pl.BlockSpec((1, P, H, W), lambda n: (n, 0, 0, 0)),
            pl.BlockSpec((1, T, H, W), lambda n: (n, 0, 0, 0)),
        ],
        out_specs=pl.BlockSpec((1, H, W), lambda n: (n, 0, 0)),
        out_shape=jax.ShapeDtypeStruct((N, H, W), jnp.float32),
        compiler_params=pltpu.CompilerParams(
            dimension_semantics=("parallel",),
            vmem_limit_bytes=56 * 1024 * 1024,
        ),
        name="cholec_metric",
    )(groups_pred, groups_true)
